# Initial kernel scaffold; baseline (speedup 1.0000x reference)
import jax
import jax.numpy as jnp
from jax.experimental import pallas as pl


def kernel(features, edge_index, W1, W2):
    raise NotImplementedError("write your pallas kernel here")


# speedup vs baseline: 1.0555x; 1.0555x over previous
import functools

import jax
import jax.numpy as jnp
from jax.experimental import pallas as pl
from jax.experimental.pallas import tpu as pltpu

N_NODES = 10000
D_FEAT = 256
H1 = 128
H2 = 64
N_EDGES = 320000

BM = 1000


def _mm1_body(x_ref, w_ref, o_ref):
    o_ref[...] = jnp.dot(x_ref[...], w_ref[...],
                         preferred_element_type=jnp.float32)


def _mid_body(h_ref, nz_ref, w_ref, o_ref):
    noisy = jnp.maximum(h_ref[...], 0.0) + nz_ref[...]
    o_ref[...] = jnp.dot(noisy, w_ref[...],
                         preferred_element_type=jnp.float32)


def _dec_body(a_ref, b_ref, o_ref):
    o_ref[...] = jax.lax.dot_general(
        a_ref[...], b_ref[...],
        (((1,), (1,)), ((), ())),
        preferred_element_type=jnp.float32)


def _mm1(x, w):
    return pl.pallas_call(
        _mm1_body,
        grid=(N_NODES // BM,),
        in_specs=[
            pl.BlockSpec((BM, D_FEAT), lambda i: (i, 0)),
            pl.BlockSpec((D_FEAT, H1), lambda i: (0, 0)),
        ],
        out_specs=pl.BlockSpec((BM, H1), lambda i: (i, 0)),
        out_shape=jax.ShapeDtypeStruct((N_NODES, H1), jnp.float32),
    )(x, w)


def _mid(h1, noise, w2):
    return pl.pallas_call(
        _mid_body,
        grid=(N_NODES // BM,),
        in_specs=[
            pl.BlockSpec((BM, H1), lambda i: (i, 0)),
            pl.BlockSpec((BM, H1), lambda i: (i, 0)),
            pl.BlockSpec((H1, H2), lambda i: (0, 0)),
        ],
        out_specs=pl.BlockSpec((BM, H2), lambda i: (i, 0)),
        out_shape=jax.ShapeDtypeStruct((N_NODES, H2), jnp.float32),
    )(h1, noise, w2)


DEC_BM = 200


def _decoder(z):
    out = pl.pallas_call(
        _dec_body,
        grid=(N_NODES // DEC_BM,),
        in_specs=[
            pl.BlockSpec((DEC_BM, H2), lambda i: (i, 0)),
            pl.BlockSpec((N_NODES, H2), lambda i: (0, 0)),
        ],
        out_specs=pl.BlockSpec((DEC_BM, N_NODES), lambda i: (i, 0)),
        out_shape=jax.ShapeDtypeStruct((N_NODES, N_NODES), jnp.float32),
    )(z, z)
    return out.reshape(-1)


def kernel(features, edge_index, W1, W2):
    src = edge_index[0]
    dst = edge_index[1]
    xw = _mm1(features, W1)
    h1 = jax.ops.segment_sum(jnp.take(xw, src, axis=0), dst,
                             num_segments=N_NODES)
    noise = 0.1 * jax.random.normal(jax.random.key(42), (N_NODES, H1),
                                    dtype=jnp.float32)
    hw = _mid(h1, noise, W2)
    z = jax.ops.segment_sum(jnp.take(hw, src, axis=0), dst,
                            num_segments=N_NODES)
    return _decoder(z)
